# baseline (speedup 1.0000x reference)
import functools

import jax
import jax.numpy as jnp
from jax import lax
from jax.experimental import pallas as pl
from jax.experimental.pallas import tpu as pltpu
from jax.experimental.pallas import tpu_sc as plsc

NUM_BINS = 100000
DIM = 32
B = 16384
N = 100

NC = 2
NS = 16
NW = NC * NS

B_PER_W = B // NW
NBB = B_PER_W // 128
NDB = DIM // 8
NBUF = 2

_mesh = plsc.VectorSubcoreMesh(core_axis_name="c", subcore_axis_name="s")


@functools.partial(
    pl.kernel,
    out_type=jax.ShapeDtypeStruct((N, NDB, B // 128, 8, 128), jnp.float32),
    mesh=_mesh,
    scratch_types=[
        pltpu.VMEM((NBUF, B_PER_W), jnp.int32),
        pltpu.VMEM((NBUF, B_PER_W, DIM), jnp.float32),
        pltpu.VMEM((NBUF, NDB, NBB, 8, 128), jnp.float32),
        [pltpu.SemaphoreType.DMA] * NBUF,
        [pltpu.SemaphoreType.DMA] * NBUF,
    ],
    compiler_params=pltpu.CompilerParams(
        use_tc_tiling_on_sc=False, needs_layout_passes=False
    ),
)
def _emb_lookup(table_hbm, xt_hbm, out_hbm, idx_v, rows_v, trans_v, gsem, osem):
    wid = lax.axis_index("s") * NC + lax.axis_index("c")
    b0 = wid * B_PER_W
    bb0 = wid * NBB
    iota16 = lax.iota(jnp.int32, 16)

    def load_idx(s, n):
        pltpu.sync_copy(xt_hbm.at[n, pl.ds(b0, B_PER_W)], idx_v.at[s])

    def fire_gathers(s):
        for j in range(NBB):
            pltpu.async_copy(
                table_hbm.at[idx_v.at[s].at[pl.ds(j * 128, 128)]],
                rows_v.at[s].at[pl.ds(j * 128, 128)],
                gsem[s],
            )

    def wait_gathers(s):
        for j in range(NBB):
            pltpu.make_async_copy(
                table_hbm.at[idx_v.at[s].at[pl.ds(j * 128, 128)]],
                rows_v.at[s].at[pl.ds(j * 128, 128)],
                gsem[s],
            ).wait()

    def transpose(s):
        dblk_v = 2 + (iota16 >> 3)
        din_v = iota16 & 7
        @plsc.parallel_loop(0, B_PER_W, unroll=4)
        def _(k):
            dblk = k >> 8
            bb = (k >> 6) & (NBB - 1)
            din = (k >> 3) & 7
            c = k & 7
            cidx = jnp.full((16,), dblk * 8 + din, jnp.int32)
            ridx = iota16 + (bb * 128 + c * 16)
            vals_g = plsc.load_gather(rows_v.at[s], [ridx, cidx])
            trans_v[s, dblk, bb, din, pl.ds(c * 16, 16)] = vals_g
            vals_s = rows_v[s, k, pl.ds(16, 16)]
            bb_v = jnp.full((16,), k >> 7, jnp.int32)
            bin_v = jnp.full((16,), k & 127, jnp.int32)
            plsc.store_scatter(trans_v.at[s], [dblk_v, bb_v, din_v, bin_v], vals_s)

    def fire_out(s, n):
        for dblk in range(NDB):
            pltpu.async_copy(
                trans_v.at[s].at[dblk],
                out_hbm.at[n, dblk, pl.ds(bb0, NBB)],
                osem[s],
            )

    def wait_out(s, n):
        for dblk in range(NDB):
            pltpu.make_async_copy(
                trans_v.at[s].at[dblk],
                out_hbm.at[n, dblk, pl.ds(bb0, NBB)],
                osem[s],
            ).wait()

    load_idx(0, 0)
    fire_gathers(0)
    load_idx(1, 1)

    def body(g, carry):
        for s in range(NBUF):
            n = g * NBUF + s
            wait_gathers(s)

            @pl.when(n + 1 < N)
            def _():
                fire_gathers(1 - s)

            @pl.when(n >= NBUF)
            def _():
                wait_out(s, n - NBUF)

            transpose(s)
            fire_out(s, n)

            @pl.when(n + NBUF < N)
            def _():
                load_idx(s, n + NBUF)

        return carry

    lax.fori_loop(0, N // NBUF, body, 0)
    wait_out(0, N - 2)
    wait_out(1, N - 1)


def kernel(x, weight):
    o5 = _emb_lookup(weight, x.T.astype(jnp.int32))
    return o5.transpose(2, 4, 0, 1, 3).reshape(B, N, DIM)


# speedup vs baseline: 2.3655x; 2.3655x over previous
import functools

import jax
import jax.numpy as jnp
from jax import lax
from jax.experimental import pallas as pl
from jax.experimental.pallas import tpu as pltpu
from jax.experimental.pallas import tpu_sc as plsc

NUM_BINS = 100000
DIM = 32
B = 16384
N = 100

NC = 2
NS = 16
NW = NC * NS

B_PER_W = B // NW
NBB = B_PER_W // 128
NDB = DIM // 8
NBUF = 2

_mesh = plsc.VectorSubcoreMesh(core_axis_name="c", subcore_axis_name="s")


@functools.partial(
    pl.kernel,
    out_type=jax.ShapeDtypeStruct((N, NDB, B // 128, 8, 128), jnp.float32),
    mesh=_mesh,
    scratch_types=[
        pltpu.VMEM((NBUF, B_PER_W), jnp.int32),
        pltpu.VMEM((NBUF, B_PER_W, DIM), jnp.float32),
        pltpu.VMEM((NBUF, NDB, NBB, 8, 128), jnp.float32),
        [pltpu.SemaphoreType.DMA] * NBUF,
        [pltpu.SemaphoreType.DMA] * NBUF,
    ],
    compiler_params=pltpu.CompilerParams(
        use_tc_tiling_on_sc=False, needs_layout_passes=False
    ),
)
def _emb_lookup(table_hbm, xt_hbm, out_hbm, idx_v, rows_v, trans_v, gsem, osem):
    wid = lax.axis_index("s") * NC + lax.axis_index("c")
    b0 = wid * B_PER_W
    bb0 = wid * NBB
    iota16 = lax.iota(jnp.int32, 16)

    def load_idx(s, n):
        pltpu.sync_copy(xt_hbm.at[n, pl.ds(b0, B_PER_W)], idx_v.at[s])

    def fire_gathers(s):
        for j in range(NBB):
            pltpu.async_copy(
                table_hbm.at[idx_v.at[s].at[pl.ds(j * 128, 128)]],
                rows_v.at[s].at[pl.ds(j * 128, 128)],
                gsem[s],
            )

    def wait_gathers(s):
        for j in range(NBB):
            pltpu.make_async_copy(
                table_hbm.at[idx_v.at[s].at[pl.ds(j * 128, 128)]],
                rows_v.at[s].at[pl.ds(j * 128, 128)],
                gsem[s],
            ).wait()

    def transpose(s):
        for dblk in range(NDB):
            for j in range(8):
                din_v = (iota16 + j) & 7
                cidx = din_v + dblk * 8
                @plsc.parallel_loop(0, B_PER_W // 16, unroll=4)
                def _(t, dblk=dblk, din_v=din_v, cidx=cidx):
                    ridx = iota16 + t * 16
                    bb_v = jnp.full((16,), t >> 3, jnp.int32)
                    bin_v = iota16 + (t & 7) * 16
                    vals = plsc.load_gather(rows_v.at[s], [ridx, cidx])
                    plsc.store_scatter(
                        trans_v.at[s].at[dblk], [bb_v, din_v, bin_v], vals
                    )

    def fire_out(s, n):
        for dblk in range(NDB):
            pltpu.async_copy(
                trans_v.at[s].at[dblk],
                out_hbm.at[n, dblk, pl.ds(bb0, NBB)],
                osem[s],
            )

    def wait_out(s, n):
        for dblk in range(NDB):
            pltpu.make_async_copy(
                trans_v.at[s].at[dblk],
                out_hbm.at[n, dblk, pl.ds(bb0, NBB)],
                osem[s],
            ).wait()

    load_idx(0, 0)
    fire_gathers(0)
    load_idx(1, 1)

    def body(g, carry):
        for s in range(NBUF):
            n = g * NBUF + s
            wait_gathers(s)

            @pl.when(n + 1 < N)
            def _():
                fire_gathers(1 - s)

            @pl.when(n >= NBUF)
            def _():
                wait_out(s, n - NBUF)

            transpose(s)
            fire_out(s, n)

            @pl.when(n + NBUF < N)
            def _():
                load_idx(s, n + NBUF)

        return carry

    lax.fori_loop(0, N // NBUF, body, 0)
    wait_out(0, N - 2)
    wait_out(1, N - 1)


def kernel(x, weight):
    o5 = _emb_lookup(weight, x.T.astype(jnp.int32))
    return o5.transpose(2, 4, 0, 1, 3).reshape(B, N, DIM)
